# baseline (speedup 1.0000x reference)
import functools

import numpy as np
import jax
import jax.numpy as jnp
from jax import lax
from jax.experimental import pallas as pl
from jax.experimental.pallas import tpu as pltpu
from jax.experimental.pallas import tpu_sc as plsc

_NEG_OV = 0.3
_POS_OV = 0.7
_STRIDE = 16
_LANES = 128
_GT_PAD = 16


def _anchor_grid(rr, cc, stride):
    w = h = float(stride)
    x_ctr = 0.5 * (w - 1.0)
    y_ctr = 0.5 * (h - 1.0)
    size = w * h
    rows = []
    for r in (0.5, 1.0, 2.0):
        ws = np.round(np.sqrt(size / r))
        hs = np.round(ws * r)
        for s in (8, 16, 32):
            wss = ws * s
            hss = hs * s
            rows.append([x_ctr - 0.5 * (wss - 1.0), y_ctr - 0.5 * (hss - 1.0),
                         x_ctr + 0.5 * (wss - 1.0), y_ctr + 0.5 * (hss - 1.0)])
    base = np.array(rows, dtype=np.float32)
    sx = np.arange(cc, dtype=np.float32) * stride
    sy = np.arange(rr, dtype=np.float32) * stride
    mx, my = np.meshgrid(sx, sy)
    shifts = np.stack([mx.ravel(), my.ravel(), mx.ravel(), my.ravel()], axis=1)
    return (base[None, :, :] + shifts[:, None, :]).reshape(-1, 4).astype(np.float32)


def _iou_body(num_gt, sc_ref, anc_ref, labels_ref, idx_ref):
    ax1 = anc_ref[0]
    ay1 = anc_ref[1]
    ax2 = anc_ref[2]
    ay2 = anc_ref[3]
    area_a = (ax2 - ax1 + 1.0) * (ay2 - ay1 + 1.0)
    m0 = sc_ref[4, 0]
    m1 = sc_ref[4, 1]
    inside = (ax1 >= 0.0) & (ay1 >= 0.0) & (ax2 < m1) & (ay2 < m0)
    shape = ax1.shape
    neg1 = jnp.full(shape, -1.0, jnp.float32)
    zero = jnp.zeros(shape, jnp.float32)

    def step(g, c):
        best, bidx, anyeq = c
        gx1 = sc_ref[0, g]
        gy1 = sc_ref[1, g]
        gx2 = sc_ref[2, g]
        gy2 = sc_ref[3, g]
        iw = jnp.maximum(jnp.minimum(ax2, gx2) - jnp.maximum(ax1, gx1) + 1.0, 0.0)
        ih = jnp.maximum(jnp.minimum(ay2, gy2) - jnp.maximum(ay1, gy1) + 1.0, 0.0)
        inter = iw * ih
        area_g = (gx2 - gx1 + 1.0) * (gy2 - gy1 + 1.0)
        ov = inter / (area_a + area_g - inter)
        upd = ov > best
        best = jnp.where(upd, ov, best)
        bidx = jnp.where(upd, g, bidx)
        masked = jnp.where(inside, ov, neg1)
        gmax = jnp.max(masked)
        anyeq = jnp.maximum(anyeq, jnp.where(masked == gmax, 1.0, 0.0))
        return best, bidx, anyeq

    init = (neg1, jnp.zeros(shape, jnp.int32), zero)
    best, bidx, anyeq = lax.fori_loop(0, num_gt, step, init)

    labels = jnp.where(best < _NEG_OV, 0.0, -1.0)
    labels = jnp.where(anyeq > 0.0, 1.0, labels)
    labels = jnp.where(best >= _POS_OV, 1.0, labels)
    labels_ref[...] = jnp.where(inside, labels, -1.0)
    idx_ref[...] = bidx


def _sc_gather_body(n, idx_hbm, tx1, ty1, tx2, ty2, out_hbm,
                    idx_v, p0, p1, p2, p3, sem):
    bpw = n // 32
    nchunk = bpw // _LANES
    wid = lax.axis_index("s") * 2 + lax.axis_index("c")
    pltpu.sync_copy(idx_hbm.at[wid], idx_v)
    planes = (p0, p1, p2, p3)
    copies = []
    for c, t in enumerate((tx1, ty1, tx2, ty2)):
        for j in range(nchunk):
            copies.append(pltpu.async_copy(
                t.at[idx_v.at[j]], planes[c].at[pl.ds(j * _LANES, _LANES)], sem))
    for cp in copies:
        cp.wait()
    base = wid * bpw
    for c in range(4):
        pltpu.sync_copy(planes[c], out_hbm.at[c, pl.ds(base, bpw)])


def _transform_body(sc_ref, anc_ref, g_ref, targets_ref):
    ax1 = anc_ref[0]
    ay1 = anc_ref[1]
    ax2 = anc_ref[2]
    ay2 = anc_ref[3]
    m0 = sc_ref[4, 0]
    m1 = sc_ref[4, 1]
    inside = (ax1 >= 0.0) & (ay1 >= 0.0) & (ax2 < m1) & (ay2 < m0)
    bx1 = g_ref[0]
    by1 = g_ref[1]
    bx2 = g_ref[2]
    by2 = g_ref[3]
    ex_w = ax2 - ax1 + 1.0
    ex_h = ay2 - ay1 + 1.0
    ex_cx = ax1 + 0.5 * ex_w
    ex_cy = ay1 + 0.5 * ex_h
    gt_w = bx2 - bx1 + 1.0
    gt_h = by2 - by1 + 1.0
    gt_cx = bx1 + 0.5 * gt_w
    gt_cy = by1 + 0.5 * gt_h
    targets_ref[0] = jnp.where(inside, (gt_cx - ex_cx) / ex_w, 0.0)
    targets_ref[1] = jnp.where(inside, (gt_cy - ex_cy) / ex_h, 0.0)
    targets_ref[2] = jnp.where(inside, jnp.log(gt_w / ex_w), 0.0)
    targets_ref[3] = jnp.where(inside, jnp.log(gt_h / ex_h), 0.0)


def kernel(scores, gt_boxes, metadata):
    rr, cc = scores.shape[1], scores.shape[2]
    anchors = _anchor_grid(rr, cc, _STRIDE)
    n = anchors.shape[0]
    num_gt = gt_boxes.shape[1]
    assert num_gt <= _LANES and n % (_LANES * 32) == 0
    rows = n // _LANES
    anc_planes = jnp.asarray(anchors.T.reshape(4, rows, _LANES))
    scalars = (jnp.zeros((5, _LANES), jnp.float32)
               .at[:4, :num_gt].set(gt_boxes[0].T)
               .at[4, :3].set(metadata[0]))

    labels, idx = pl.pallas_call(
        functools.partial(_iou_body, num_gt),
        out_shape=[
            jax.ShapeDtypeStruct((rows, _LANES), jnp.float32),
            jax.ShapeDtypeStruct((rows, _LANES), jnp.int32),
        ],
        in_specs=[
            pl.BlockSpec(memory_space=pltpu.SMEM),
            pl.BlockSpec(memory_space=pltpu.VMEM),
        ],
    )(scalars, anc_planes)

    bpw = n // 32
    nchunk = bpw // _LANES
    sc_gather = functools.partial(
        pl.kernel,
        mesh=plsc.VectorSubcoreMesh(core_axis_name="c", subcore_axis_name="s"),
        out_type=jax.ShapeDtypeStruct((4, n), jnp.float32),
        scratch_types=[
            pltpu.VMEM((nchunk, _LANES), jnp.int32),
            pltpu.VMEM((bpw,), jnp.float32),
            pltpu.VMEM((bpw,), jnp.float32),
            pltpu.VMEM((bpw,), jnp.float32),
            pltpu.VMEM((bpw,), jnp.float32),
            pltpu.SemaphoreType.DMA,
        ],
    )(functools.partial(_sc_gather_body, n))
    gathered = sc_gather(idx.reshape(32, nchunk, _LANES),
                         gt_boxes[0, :, 0], gt_boxes[0, :, 1],
                         gt_boxes[0, :, 2], gt_boxes[0, :, 3])

    targets = pl.pallas_call(
        _transform_body,
        out_shape=jax.ShapeDtypeStruct((4, rows, _LANES), jnp.float32),
        in_specs=[
            pl.BlockSpec(memory_space=pltpu.SMEM),
            pl.BlockSpec(memory_space=pltpu.VMEM),
            pl.BlockSpec(memory_space=pltpu.VMEM),
        ],
    )(scalars, anc_planes, gathered.reshape(4, rows, _LANES))

    return labels.reshape(1, n), targets.reshape(4, n).T[None]


# speedup vs baseline: 1.0061x; 1.0061x over previous
import functools

import numpy as np
import jax
import jax.numpy as jnp
from jax import lax
from jax.experimental import pallas as pl
from jax.experimental.pallas import tpu as pltpu
from jax.experimental.pallas import tpu_sc as plsc

_NEG_OV = 0.3
_POS_OV = 0.7
_STRIDE = 16
_LANES = 128
_GT_PAD = 16


def _anchor_grid(rr, cc, stride):
    w = h = float(stride)
    x_ctr = 0.5 * (w - 1.0)
    y_ctr = 0.5 * (h - 1.0)
    size = w * h
    rows = []
    for r in (0.5, 1.0, 2.0):
        ws = np.round(np.sqrt(size / r))
        hs = np.round(ws * r)
        for s in (8, 16, 32):
            wss = ws * s
            hss = hs * s
            rows.append([x_ctr - 0.5 * (wss - 1.0), y_ctr - 0.5 * (hss - 1.0),
                         x_ctr + 0.5 * (wss - 1.0), y_ctr + 0.5 * (hss - 1.0)])
    base = np.array(rows, dtype=np.float32)
    sx = np.arange(cc, dtype=np.float32) * stride
    sy = np.arange(rr, dtype=np.float32) * stride
    mx, my = np.meshgrid(sx, sy)
    shifts = np.stack([mx.ravel(), my.ravel(), mx.ravel(), my.ravel()], axis=1)
    return (base[None, :, :] + shifts[:, None, :]).reshape(-1, 4).astype(np.float32)


def _iou_body(num_gt, sc_ref, anc_ref, labels_ref, idx_ref):
    ax1 = anc_ref[0]
    ay1 = anc_ref[1]
    ax2 = anc_ref[2]
    ay2 = anc_ref[3]
    area_a = (ax2 - ax1 + 1.0) * (ay2 - ay1 + 1.0)
    m0 = sc_ref[4, 0]
    m1 = sc_ref[4, 1]
    inside = (ax1 >= 0.0) & (ay1 >= 0.0) & (ax2 < m1) & (ay2 < m0)
    shape = ax1.shape
    neg1 = jnp.full(shape, -1.0, jnp.float32)
    zero = jnp.zeros(shape, jnp.float32)

    def step(g, c):
        best, bidx, anyeq = c
        gx1 = sc_ref[0, g]
        gy1 = sc_ref[1, g]
        gx2 = sc_ref[2, g]
        gy2 = sc_ref[3, g]
        iw = jnp.maximum(jnp.minimum(ax2, gx2) - jnp.maximum(ax1, gx1) + 1.0, 0.0)
        ih = jnp.maximum(jnp.minimum(ay2, gy2) - jnp.maximum(ay1, gy1) + 1.0, 0.0)
        inter = iw * ih
        area_g = (gx2 - gx1 + 1.0) * (gy2 - gy1 + 1.0)
        ov = inter / (area_a + area_g - inter)
        upd = ov > best
        best = jnp.where(upd, ov, best)
        bidx = jnp.where(upd, g, bidx)
        masked = jnp.where(inside, ov, neg1)
        gmax = jnp.max(masked)
        anyeq = jnp.maximum(anyeq, jnp.where(masked == gmax, 1.0, 0.0))
        return best, bidx, anyeq

    init = (neg1, jnp.zeros(shape, jnp.int32), zero)
    best, bidx, anyeq = lax.fori_loop(0, num_gt, step, init)

    labels = jnp.where(best < _NEG_OV, 0.0, -1.0)
    labels = jnp.where(anyeq > 0.0, 1.0, labels)
    labels = jnp.where(best >= _POS_OV, 1.0, labels)
    labels_ref[...] = jnp.where(inside, labels, -1.0)
    idx_ref[...] = bidx


def _sc_gather_body(idx_hbm, tx1, ty1, tx2, ty2, out_hbm,
                    idx_v, p0, p1, p2, p3, sem):
    wid = lax.axis_index("s") * 2 + lax.axis_index("c")
    pltpu.sync_copy(idx_hbm.at[wid], idx_v)
    planes = (p0, p1, p2, p3)
    copies = [pltpu.async_copy(t.at[idx_v], planes[c], sem)
              for c, t in enumerate((tx1, ty1, tx2, ty2))]
    for cp in copies:
        cp.wait()
    for c in range(4):
        pltpu.sync_copy(planes[c], out_hbm.at[c, wid])


def _transform_body(sc_ref, anc_ref, g_ref, targets_ref):
    ax1 = anc_ref[0]
    ay1 = anc_ref[1]
    ax2 = anc_ref[2]
    ay2 = anc_ref[3]
    m0 = sc_ref[4, 0]
    m1 = sc_ref[4, 1]
    inside = (ax1 >= 0.0) & (ay1 >= 0.0) & (ax2 < m1) & (ay2 < m0)
    bx1 = g_ref[0]
    by1 = g_ref[1]
    bx2 = g_ref[2]
    by2 = g_ref[3]
    ex_w = ax2 - ax1 + 1.0
    ex_h = ay2 - ay1 + 1.0
    ex_cx = ax1 + 0.5 * ex_w
    ex_cy = ay1 + 0.5 * ex_h
    gt_w = bx2 - bx1 + 1.0
    gt_h = by2 - by1 + 1.0
    gt_cx = bx1 + 0.5 * gt_w
    gt_cy = by1 + 0.5 * gt_h
    targets_ref[0] = jnp.where(inside, (gt_cx - ex_cx) / ex_w, 0.0)
    targets_ref[1] = jnp.where(inside, (gt_cy - ex_cy) / ex_h, 0.0)
    targets_ref[2] = jnp.where(inside, jnp.log(gt_w / ex_w), 0.0)
    targets_ref[3] = jnp.where(inside, jnp.log(gt_h / ex_h), 0.0)


def kernel(scores, gt_boxes, metadata):
    rr, cc = scores.shape[1], scores.shape[2]
    anchors = _anchor_grid(rr, cc, _STRIDE)
    n = anchors.shape[0]
    num_gt = gt_boxes.shape[1]
    assert num_gt <= _LANES and n % (_LANES * 32) == 0
    rows = n // _LANES
    anc_planes = jnp.asarray(anchors.T.reshape(4, rows, _LANES))
    scalars = (jnp.zeros((5, _LANES), jnp.float32)
               .at[:4, :num_gt].set(gt_boxes[0].T)
               .at[4, :3].set(metadata[0]))

    labels, idx = pl.pallas_call(
        functools.partial(_iou_body, num_gt),
        out_shape=[
            jax.ShapeDtypeStruct((rows, _LANES), jnp.float32),
            jax.ShapeDtypeStruct((rows, _LANES), jnp.int32),
        ],
        in_specs=[
            pl.BlockSpec(memory_space=pltpu.SMEM),
            pl.BlockSpec(memory_space=pltpu.VMEM),
        ],
    )(scalars, anc_planes)

    bpw = n // 32
    nchunk = bpw // _LANES
    sc_gather = functools.partial(
        pl.kernel,
        mesh=plsc.VectorSubcoreMesh(core_axis_name="c", subcore_axis_name="s"),
        out_type=jax.ShapeDtypeStruct((4, 32, bpw), jnp.float32),
        scratch_types=[
            pltpu.VMEM((bpw,), jnp.int32),
            pltpu.VMEM((bpw,), jnp.float32),
            pltpu.VMEM((bpw,), jnp.float32),
            pltpu.VMEM((bpw,), jnp.float32),
            pltpu.VMEM((bpw,), jnp.float32),
            pltpu.SemaphoreType.DMA,
        ],
    )(_sc_gather_body)
    gathered = sc_gather(idx.reshape(32, bpw),
                         gt_boxes[0, :, 0], gt_boxes[0, :, 1],
                         gt_boxes[0, :, 2], gt_boxes[0, :, 3])

    targets = pl.pallas_call(
        _transform_body,
        out_shape=jax.ShapeDtypeStruct((4, rows, _LANES), jnp.float32),
        in_specs=[
            pl.BlockSpec(memory_space=pltpu.SMEM),
            pl.BlockSpec(memory_space=pltpu.VMEM),
            pl.BlockSpec(memory_space=pltpu.VMEM),
        ],
    )(scalars, anc_planes, gathered.reshape(4, rows, _LANES))

    return labels.reshape(1, n), targets.reshape(4, n).T[None]


# speedup vs baseline: 8.3271x; 8.2765x over previous
import functools

import numpy as np
import jax
import jax.numpy as jnp
from jax import lax
from jax.experimental import pallas as pl
from jax.experimental.pallas import tpu as pltpu
from jax.experimental.pallas import tpu_sc as plsc

_NEG_OV = 0.3
_POS_OV = 0.7
_STRIDE = 16
_LANES = 128
_GT_PAD = 16


def _anchor_grid(rr, cc, stride):
    w = h = float(stride)
    x_ctr = 0.5 * (w - 1.0)
    y_ctr = 0.5 * (h - 1.0)
    size = w * h
    rows = []
    for r in (0.5, 1.0, 2.0):
        ws = np.round(np.sqrt(size / r))
        hs = np.round(ws * r)
        for s in (8, 16, 32):
            wss = ws * s
            hss = hs * s
            rows.append([x_ctr - 0.5 * (wss - 1.0), y_ctr - 0.5 * (hss - 1.0),
                         x_ctr + 0.5 * (wss - 1.0), y_ctr + 0.5 * (hss - 1.0)])
    base = np.array(rows, dtype=np.float32)
    sx = np.arange(cc, dtype=np.float32) * stride
    sy = np.arange(rr, dtype=np.float32) * stride
    mx, my = np.meshgrid(sx, sy)
    shifts = np.stack([mx.ravel(), my.ravel(), mx.ravel(), my.ravel()], axis=1)
    return (base[None, :, :] + shifts[:, None, :]).reshape(-1, 4).astype(np.float32)


def _iou_body(num_gt, sc_ref, anc_ref, labels_ref, idx_ref):
    ax1 = anc_ref[0]
    ay1 = anc_ref[1]
    ax2 = anc_ref[2]
    ay2 = anc_ref[3]
    area_a = (ax2 - ax1 + 1.0) * (ay2 - ay1 + 1.0)
    m0 = sc_ref[4, 0]
    m1 = sc_ref[4, 1]
    inside = (ax1 >= 0.0) & (ay1 >= 0.0) & (ax2 < m1) & (ay2 < m0)
    shape = ax1.shape
    neg1 = jnp.full(shape, -1.0, jnp.float32)
    zero = jnp.zeros(shape, jnp.float32)

    def step(g, c):
        best, bidx, anyeq = c
        gx1 = sc_ref[0, g]
        gy1 = sc_ref[1, g]
        gx2 = sc_ref[2, g]
        gy2 = sc_ref[3, g]
        iw = jnp.maximum(jnp.minimum(ax2, gx2) - jnp.maximum(ax1, gx1) + 1.0, 0.0)
        ih = jnp.maximum(jnp.minimum(ay2, gy2) - jnp.maximum(ay1, gy1) + 1.0, 0.0)
        inter = iw * ih
        area_g = (gx2 - gx1 + 1.0) * (gy2 - gy1 + 1.0)
        ov = inter / (area_a + area_g - inter)
        upd = ov > best
        best = jnp.where(upd, ov, best)
        bidx = jnp.where(upd, g, bidx)
        masked = jnp.where(inside, ov, neg1)
        gmax = jnp.max(masked)
        anyeq = jnp.maximum(anyeq, jnp.where(masked == gmax, 1.0, 0.0))
        return best, bidx, anyeq

    init = (neg1, jnp.zeros(shape, jnp.int32), zero)
    best, bidx, anyeq = lax.fori_loop(0, num_gt, step, init)

    labels = jnp.where(best < _NEG_OV, 0.0, -1.0)
    labels = jnp.where(anyeq > 0.0, 1.0, labels)
    labels = jnp.where(best >= _POS_OV, 1.0, labels)
    labels_ref[...] = jnp.where(inside, labels, -1.0)
    idx_ref[...] = bidx


def _sc_gather_body(bpw, gt_pad, idx_hbm, tx1, ty1, tx2, ty2, out_hbm,
                    idx_v, t0, t1, t2, t3, p0, p1, p2, p3):
    wid = lax.axis_index("s") * 2 + lax.axis_index("c")
    pltpu.sync_copy(idx_hbm.at[wid], idx_v)
    tables = (t0, t1, t2, t3)
    planes = (p0, p1, p2, p3)
    for c, t in enumerate((tx1, ty1, tx2, ty2)):
        pltpu.sync_copy(t, tables[c])

    def body(j, carry):
        j0 = j * 16
        idx16 = idx_v[pl.ds(j0, 16)]
        for c in range(4):
            acc = jnp.zeros((16,), jnp.float32)
            for t in range(gt_pad // 16):
                tv = tables[c][pl.ds(t * 16, 16)]
                local = idx16 - t * 16
                valid = (local >= 0) & (local < 16)
                safe = jnp.where(valid, local, 0)
                g = tv.at[safe].get(mode="promise_in_bounds")
                acc = jnp.where(valid, g, acc)
            planes[c][pl.ds(j0, 16)] = acc
        return carry

    lax.fori_loop(0, bpw // 16, body, 0)
    for c in range(4):
        pltpu.sync_copy(planes[c], out_hbm.at[c, wid])


def _transform_body(sc_ref, anc_ref, g_ref, targets_ref):
    ax1 = anc_ref[0]
    ay1 = anc_ref[1]
    ax2 = anc_ref[2]
    ay2 = anc_ref[3]
    m0 = sc_ref[4, 0]
    m1 = sc_ref[4, 1]
    inside = (ax1 >= 0.0) & (ay1 >= 0.0) & (ax2 < m1) & (ay2 < m0)
    bx1 = g_ref[0]
    by1 = g_ref[1]
    bx2 = g_ref[2]
    by2 = g_ref[3]
    ex_w = ax2 - ax1 + 1.0
    ex_h = ay2 - ay1 + 1.0
    ex_cx = ax1 + 0.5 * ex_w
    ex_cy = ay1 + 0.5 * ex_h
    gt_w = bx2 - bx1 + 1.0
    gt_h = by2 - by1 + 1.0
    gt_cx = bx1 + 0.5 * gt_w
    gt_cy = by1 + 0.5 * gt_h
    targets_ref[0] = jnp.where(inside, (gt_cx - ex_cx) / ex_w, 0.0)
    targets_ref[1] = jnp.where(inside, (gt_cy - ex_cy) / ex_h, 0.0)
    targets_ref[2] = jnp.where(inside, jnp.log(gt_w / ex_w), 0.0)
    targets_ref[3] = jnp.where(inside, jnp.log(gt_h / ex_h), 0.0)


def kernel(scores, gt_boxes, metadata):
    rr, cc = scores.shape[1], scores.shape[2]
    anchors = _anchor_grid(rr, cc, _STRIDE)
    n = anchors.shape[0]
    num_gt = gt_boxes.shape[1]
    assert num_gt <= _LANES and n % (_LANES * 32) == 0
    rows = n // _LANES
    anc_planes = jnp.asarray(anchors.T.reshape(4, rows, _LANES))
    scalars = (jnp.zeros((5, _LANES), jnp.float32)
               .at[:4, :num_gt].set(gt_boxes[0].T)
               .at[4, :3].set(metadata[0]))

    labels, idx = pl.pallas_call(
        functools.partial(_iou_body, num_gt),
        out_shape=[
            jax.ShapeDtypeStruct((rows, _LANES), jnp.float32),
            jax.ShapeDtypeStruct((rows, _LANES), jnp.int32),
        ],
        in_specs=[
            pl.BlockSpec(memory_space=pltpu.SMEM),
            pl.BlockSpec(memory_space=pltpu.VMEM),
        ],
    )(scalars, anc_planes)

    bpw = n // 32
    gt_pad = ((num_gt + 15) // 16) * 16
    sc_gather = functools.partial(
        pl.kernel,
        mesh=plsc.VectorSubcoreMesh(core_axis_name="c", subcore_axis_name="s"),
        out_type=jax.ShapeDtypeStruct((4, 32, bpw), jnp.float32),
        scratch_types=[
            pltpu.VMEM((bpw,), jnp.int32),
            pltpu.VMEM((gt_pad,), jnp.float32),
            pltpu.VMEM((gt_pad,), jnp.float32),
            pltpu.VMEM((gt_pad,), jnp.float32),
            pltpu.VMEM((gt_pad,), jnp.float32),
            pltpu.VMEM((bpw,), jnp.float32),
            pltpu.VMEM((bpw,), jnp.float32),
            pltpu.VMEM((bpw,), jnp.float32),
            pltpu.VMEM((bpw,), jnp.float32),
        ],
    )(functools.partial(_sc_gather_body, bpw, gt_pad))
    gt_cols = jnp.zeros((4, gt_pad), jnp.float32).at[:, :num_gt].set(gt_boxes[0].T)
    gathered = sc_gather(idx.reshape(32, bpw),
                         gt_cols[0], gt_cols[1], gt_cols[2], gt_cols[3])

    targets = pl.pallas_call(
        _transform_body,
        out_shape=jax.ShapeDtypeStruct((4, rows, _LANES), jnp.float32),
        in_specs=[
            pl.BlockSpec(memory_space=pltpu.SMEM),
            pl.BlockSpec(memory_space=pltpu.VMEM),
            pl.BlockSpec(memory_space=pltpu.VMEM),
        ],
    )(scalars, anc_planes, gathered.reshape(4, rows, _LANES))

    return labels.reshape(1, n), targets.reshape(4, n).T[None]
